# baseline (speedup 1.0000x reference)
import jax
import jax.numpy as jnp
from jax.experimental import pallas as pl
from jax.experimental.pallas import tpu as pltpu

_SCHEDULE = (256, 256, 512, 1024, 2048, 2048, 2048)
_MAX_CHUNK = 2048
_NBUF = 4
_PREFETCH = 2


def kernel(x, table):
    del x
    max_len, emb_dim = table.shape
    assert sum(_SCHEDULE) == max_len
    offs = [0]
    for c in _SCHEDULE:
        offs.append(offs[-1] + c)
    nch = len(_SCHEDULE)

    def body(in_hbm, out_hbm, buf, *sems):
        sin = sems[:_NBUF]
        sout = sems[_NBUF:]

        def cin(i):
            return pltpu.make_async_copy(
                in_hbm.at[pl.ds(offs[i], _SCHEDULE[i])],
                buf.at[i % _NBUF, pl.ds(0, _SCHEDULE[i])],
                sin[i % _NBUF],
            )

        def cout(i):
            return pltpu.make_async_copy(
                buf.at[i % _NBUF, pl.ds(0, _SCHEDULE[i])],
                out_hbm.at[pl.ds(offs[i], _SCHEDULE[i])],
                sout[i % _NBUF],
            )

        for i in range(min(_PREFETCH, nch)):
            cin(i).start()
        for i in range(nch):
            cin(i).wait()
            cout(i).start()
            j = i + _PREFETCH
            if j < nch:
                if j >= _NBUF:
                    cout(j - _NBUF).wait()
                cin(j).start()
        for i in range(max(nch - _NBUF, 0), nch):
            cout(i).wait()

    out = pl.pallas_call(
        body,
        in_specs=[pl.BlockSpec(memory_space=pltpu.MemorySpace.HBM)],
        out_specs=pl.BlockSpec(memory_space=pltpu.MemorySpace.HBM),
        out_shape=jax.ShapeDtypeStruct((max_len, emb_dim), table.dtype),
        scratch_shapes=[pltpu.VMEM((_NBUF, _MAX_CHUNK, emb_dim), table.dtype)]
        + [pltpu.SemaphoreType.DMA] * (2 * _NBUF),
    )(table)
    return out[None]


# speedup vs baseline: 1.0606x; 1.0606x over previous
import jax
import jax.numpy as jnp
from jax.experimental import pallas as pl
from jax.experimental.pallas import tpu as pltpu

_SCHEDULE = (2048,) * 4
_MAX_CHUNK = 2048
_NBUF = 4
_PREFETCH = 3


def kernel(x, table):
    del x
    max_len, emb_dim = table.shape
    assert sum(_SCHEDULE) == max_len
    offs = [0]
    for c in _SCHEDULE:
        offs.append(offs[-1] + c)
    nch = len(_SCHEDULE)

    def body(in_hbm, out_hbm, buf, *sems):
        sin = sems[:_NBUF]
        sout = sems[_NBUF:]

        def cin(i):
            return pltpu.make_async_copy(
                in_hbm.at[pl.ds(offs[i], _SCHEDULE[i])],
                buf.at[i % _NBUF, pl.ds(0, _SCHEDULE[i])],
                sin[i % _NBUF],
            )

        def cout(i):
            return pltpu.make_async_copy(
                buf.at[i % _NBUF, pl.ds(0, _SCHEDULE[i])],
                out_hbm.at[pl.ds(offs[i], _SCHEDULE[i])],
                sout[i % _NBUF],
            )

        for i in range(min(_PREFETCH, nch)):
            cin(i).start()
        for i in range(nch):
            cin(i).wait()
            cout(i).start()
            j = i + _PREFETCH
            if j < nch:
                if j >= _NBUF:
                    cout(j - _NBUF).wait()
                cin(j).start()
        for i in range(max(nch - _NBUF, 0), nch):
            cout(i).wait()

    out = pl.pallas_call(
        body,
        in_specs=[pl.BlockSpec(memory_space=pltpu.MemorySpace.HBM)],
        out_specs=pl.BlockSpec(memory_space=pltpu.MemorySpace.HBM),
        out_shape=jax.ShapeDtypeStruct((max_len, emb_dim), table.dtype),
        scratch_shapes=[pltpu.VMEM((_NBUF, _MAX_CHUNK, emb_dim), table.dtype)]
        + [pltpu.SemaphoreType.DMA] * (2 * _NBUF),
    )(table)
    return out[None]
